# baseline (speedup 1.0000x reference)
import jax
import jax.numpy as jnp
from jax import lax
from jax.experimental import pallas as pl
from jax.experimental.pallas import tpu as pltpu, tpu_sc as plsc

BATCH = 4
SEQ_LEN = 8192
EMBED_DIM = 256

NB_SC = 2
NC = 2
NS = 16
LANES = 16

NW = NC * NS
ROWS_PER_W = SEQ_LEN // NW
CHUNK_ROWS = 64
CHUNKS_PER_BATCH = ROWS_PER_W // CHUNK_ROWS
NBUF = 3
N_CHUNKS = NB_SC * CHUNKS_PER_BATCH
VECS_PER_ROW = EMBED_DIM // LANES

TC_BLOCK_ROWS = 512


def _sc_body(in_hbm, pos_hbm, out_hbm, pos_v, bufs, sem_pos, sems_in, sems_out):
    wid = lax.axis_index("s") * NC + lax.axis_index("c")
    s_base = wid * ROWS_PER_W

    cp_pos = pltpu.make_async_copy(
        pos_hbm.at[pl.ds(s_base, ROWS_PER_W), :], pos_v, sem_pos)
    cp_pos.start()

    def in_cp(k, slot):
        b, piece = divmod(k, CHUNKS_PER_BATCH)
        s0 = s_base + piece * CHUNK_ROWS
        return pltpu.make_async_copy(
            in_hbm.at[b, pl.ds(s0, CHUNK_ROWS), :], bufs[slot], sems_in[slot])

    def out_cp(k, slot):
        b, piece = divmod(k, CHUNKS_PER_BATCH)
        s0 = s_base + piece * CHUNK_ROWS
        return pltpu.make_async_copy(
            bufs[slot], out_hbm.at[b, pl.ds(s0, CHUNK_ROWS), :], sems_out[slot])

    for k in range(min(NBUF - 1, N_CHUNKS)):
        in_cp(k, k % NBUF).start()

    cp_pos.wait()

    for k in range(N_CHUNKS):
        slot = k % NBUF
        nk = k + NBUF - 1
        if nk < N_CHUNKS:
            nslot = nk % NBUF
            if nk >= NBUF:
                out_cp(nk - NBUF, nslot).wait()
            in_cp(nk, nslot).start()
        in_cp(k, slot).wait()

        row0 = (k % CHUNKS_PER_BATCH) * CHUNK_ROWS
        buf = bufs[slot]

        @plsc.parallel_loop(0, CHUNK_ROWS, step=1, unroll=2)
        def _add_row(r, buf=buf, row0=row0):
            for c in range(VECS_PER_ROW):
                x = pos_v[row0 + r, pl.ds(c * LANES, LANES)]
                plsc.addupdate(buf.at[r, pl.ds(c * LANES, LANES)], x)

        out_cp(k, slot).start()

    for k in range(max(0, N_CHUNKS - NBUF), N_CHUNKS):
        out_cp(k, k % NBUF).wait()


def _sc_add(inputs_sc, pos_table):
    mesh = plsc.VectorSubcoreMesh(core_axis_name="c", subcore_axis_name="s")
    return pl.kernel(
        _sc_body,
        out_type=jax.ShapeDtypeStruct((NB_SC, SEQ_LEN, EMBED_DIM),
                                      jnp.float32),
        mesh=mesh,
        scratch_types=[
            pltpu.VMEM((ROWS_PER_W, EMBED_DIM), jnp.float32),
            [pltpu.VMEM((CHUNK_ROWS, EMBED_DIM), jnp.float32)
             for _ in range(NBUF)],
            pltpu.SemaphoreType.DMA,
            [pltpu.SemaphoreType.DMA for _ in range(NBUF)],
            [pltpu.SemaphoreType.DMA for _ in range(NBUF)],
        ],
    )(inputs_sc, pos_table)


def _tc_body(in_ref, pos_ref, out_ref):
    out_ref[...] = in_ref[...] + pos_ref[...]


def _tc_add(inputs_tc, pos_table):
    nb = BATCH - NB_SC
    grid = (nb, SEQ_LEN // TC_BLOCK_ROWS)
    return pl.pallas_call(
        _tc_body,
        grid=grid,
        in_specs=[
            pl.BlockSpec((1, TC_BLOCK_ROWS, EMBED_DIM),
                         lambda b, i: (b, i, 0)),
            pl.BlockSpec((TC_BLOCK_ROWS, EMBED_DIM), lambda b, i: (i, 0)),
        ],
        out_specs=pl.BlockSpec((1, TC_BLOCK_ROWS, EMBED_DIM),
                               lambda b, i: (b, i, 0)),
        out_shape=jax.ShapeDtypeStruct((nb, SEQ_LEN, EMBED_DIM), jnp.float32),
    )(inputs_tc, pos_table)


@jax.jit
def _pos_add(inputs, pos_table):
    out_sc = _sc_add(inputs[:NB_SC], pos_table)
    out_tc = _tc_add(inputs[NB_SC:], pos_table)
    return jnp.concatenate([out_sc, out_tc], axis=0)


def kernel(inputs, pos_table):
    return _pos_add(inputs, pos_table)


# speedup vs baseline: 1.7280x; 1.7280x over previous
import jax
import jax.numpy as jnp
from jax import lax
from jax.experimental import pallas as pl
from jax.experimental.pallas import tpu as pltpu, tpu_sc as plsc

BATCH = 4
SEQ_LEN = 8192
EMBED_DIM = 256

NC = 2
NS = 16
LANES = 16

NW = NC * NS
ROWS_PER_W = SEQ_LEN // NW
CHUNK_ROWS = 128
N_PIECES = ROWS_PER_W // CHUNK_ROWS
NBUF = 2
VECS_PER_ROW = EMBED_DIM // LANES


def _body(in_hbm, pos_hbm, out_hbm, pos_v, bufs, sem_pos, sems_in, sems_out):
    wid = lax.axis_index("s") * NC + lax.axis_index("c")
    s_base = wid * ROWS_PER_W

    def pos_cp(piece):
        s0 = s_base + piece * CHUNK_ROWS
        return pltpu.make_async_copy(
            pos_hbm.at[pl.ds(s0, CHUNK_ROWS), :], pos_v, sem_pos)

    def in_cp(k, slot):
        piece, b = divmod(k, BATCH)
        s0 = s_base + piece * CHUNK_ROWS
        return pltpu.make_async_copy(
            in_hbm.at[b, pl.ds(s0, CHUNK_ROWS), :], bufs[slot], sems_in[slot])

    def out_cp(k, slot):
        piece, b = divmod(k, BATCH)
        s0 = s_base + piece * CHUNK_ROWS
        return pltpu.make_async_copy(
            bufs[slot], out_hbm.at[b, pl.ds(s0, CHUNK_ROWS), :], sems_out[slot])

    N_CHUNKS = N_PIECES * BATCH

    pos_cp(0).start()
    for k in range(NBUF - 1):
        in_cp(k, k % NBUF).start()
    pos_cp(0).wait()

    for k in range(N_CHUNKS):
        slot = k % NBUF
        nk = k + NBUF - 1
        if nk < N_CHUNKS:
            nslot = nk % NBUF
            if nk >= NBUF:
                out_cp(nk - NBUF, nslot).wait()
            in_cp(nk, nslot).start()
        in_cp(k, slot).wait()
        if k % BATCH == BATCH - 1 and k + 1 < N_CHUNKS:
            pass

        buf = bufs[slot]

        @plsc.parallel_loop(0, CHUNK_ROWS, step=1, unroll=2)
        def _add_row(r, buf=buf):
            for c in range(VECS_PER_ROW):
                x = pos_v[r, pl.ds(c * LANES, LANES)]
                plsc.addupdate(buf.at[r, pl.ds(c * LANES, LANES)], x)

        out_cp(k, slot).start()

        if k % BATCH == BATCH - 1 and k + 1 < N_CHUNKS:
            nxt = k // BATCH + 1
            pos_cp(nxt).start()
            pos_cp(nxt).wait()

    for k in range(max(0, N_CHUNKS - NBUF), N_CHUNKS):
        out_cp(k, k % NBUF).wait()


@jax.jit
def _pos_add(inputs, pos_table):
    mesh = plsc.VectorSubcoreMesh(core_axis_name="c", subcore_axis_name="s")
    return pl.kernel(
        _body,
        out_type=jax.ShapeDtypeStruct((BATCH, SEQ_LEN, EMBED_DIM), jnp.float32),
        mesh=mesh,
        scratch_types=[
            pltpu.VMEM((CHUNK_ROWS, EMBED_DIM), jnp.float32),
            [pltpu.VMEM((CHUNK_ROWS, EMBED_DIM), jnp.float32)
             for _ in range(NBUF)],
            pltpu.SemaphoreType.DMA,
            [pltpu.SemaphoreType.DMA for _ in range(NBUF)],
            [pltpu.SemaphoreType.DMA for _ in range(NBUF)],
        ],
    )(inputs, pos_table)


def kernel(inputs, pos_table):
    return _pos_add(inputs, pos_table)
